# baseline (speedup 1.0000x reference)
import functools

import jax
import jax.numpy as jnp
from jax import lax
from jax.experimental import pallas as pl
from jax.experimental.pallas import tpu as pltpu
from jax.experimental.pallas import tpu_sc as plsc

_D = 1024
_B = 4
_S = 2048
_P = 20
_R = _S + _P

_NC = 2
_NS = 16
_NW = _NC * _NS
_TOK_W = (_B * _S) // _NW
_CHUNK = 32
_NCHUNK = _TOK_W // _CHUNK
_WPB = _NW // _B


def _embed_concat(tokens_flat, soft_prompts, W):
    mesh = plsc.VectorSubcoreMesh(core_axis_name="c", subcore_axis_name="s")

    @functools.partial(
        pl.kernel,
        mesh=mesh,
        out_type=jax.ShapeDtypeStruct((_B * _R, _D), jnp.float32),
        scratch_types=[
            pltpu.VMEM((_TOK_W,), jnp.int32),
            pltpu.VMEM((_CHUNK, _D), jnp.float32),
            pltpu.VMEM((_P, _D), jnp.float32),
            pltpu.SemaphoreType.DMA,
        ],
        compiler_params=pltpu.CompilerParams(use_tc_tiling_on_sc=False),
    )
    def k(tok_hbm, sp_hbm, w_hbm, out_hbm, idx_v, rows_v, sp_v, gsem):
        wid = lax.axis_index("s") * _NC + lax.axis_index("c")
        b = wid // _WPB
        kk = wid % _WPB
        tok_base = wid * _TOK_W
        out_base = b * _R + _P + kk * _TOK_W

        pltpu.sync_copy(tok_hbm.at[pl.ds(tok_base, _TOK_W)], idx_v)

        @pl.when(kk == 0)
        def _():
            pltpu.sync_copy(sp_hbm.at[b], sp_v)
            pltpu.sync_copy(sp_v, out_hbm.at[pl.ds(b * _R, _P)])

        def body(i, carry):
            off = pl.multiple_of(i * _CHUNK, 8)
            pltpu.async_copy(
                w_hbm.at[idx_v.at[pl.ds(off, _CHUNK)]], rows_v, gsem
            ).wait()
            pltpu.sync_copy(
                rows_v, out_hbm.at[pl.ds(out_base + i * _CHUNK, _CHUNK)]
            )
            return carry

        lax.fori_loop(0, _NCHUNK, body, 0)

    return k(tokens_flat, soft_prompts, W)


def kernel(tokens, soft_prompts, W):
    tokens_flat = tokens.reshape(-1).astype(jnp.int32)
    out_flat = _embed_concat(tokens_flat, soft_prompts, W)
    return out_flat.reshape(_B, _R, _D)


# speedup vs baseline: 3.6468x; 3.6468x over previous
import functools

import jax
import jax.numpy as jnp
from jax import lax
from jax.experimental import pallas as pl
from jax.experimental.pallas import tpu as pltpu
from jax.experimental.pallas import tpu_sc as plsc

_D = 1024
_B = 4
_S = 2048
_P = 20
_R = _S + _P

_NC = 2
_NS = 16
_NW = _NC * _NS
_TOK_W = (_B * _S) // _NW
_CHUNK = 32
_NCHUNK = _TOK_W // _CHUNK
_WPB = _NW // _B


def _embed_concat(tokens_flat, soft_prompts, W):
    mesh = plsc.VectorSubcoreMesh(core_axis_name="c", subcore_axis_name="s")

    @functools.partial(
        pl.kernel,
        mesh=mesh,
        out_type=jax.ShapeDtypeStruct((_B, _R, _D), jnp.float32),
        scratch_types=[
            pltpu.VMEM((_TOK_W,), jnp.int32),
            pltpu.VMEM((2, _CHUNK, _D), jnp.float32),
            pltpu.VMEM((24, _D), jnp.float32),
            pltpu.SemaphoreType.DMA,
            pltpu.SemaphoreType.DMA,
            pltpu.SemaphoreType.DMA,
            pltpu.SemaphoreType.DMA,
            pltpu.SemaphoreType.DMA,
        ],
    )
    def k(tok_hbm, sp_hbm, w_hbm, out_hbm, idx_v, rows_v, sp_v,
          gsem0, gsem1, osem0, osem1, psem):
        wid = lax.axis_index("s") * _NC + lax.axis_index("c")
        b = wid // _WPB
        kk = wid % _WPB
        tok_base = pl.multiple_of(wid * _TOK_W, _TOK_W)
        out_b = out_hbm.at[b]
        iota16 = lax.iota(jnp.int32, 16)

        pltpu.sync_copy(tok_hbm.at[pl.ds(tok_base, _TOK_W)], idx_v)

        @pl.when(kk == 0)
        def _():
            sp0 = _P * b
            w0 = pl.multiple_of((sp0 // 8) * 8, 8)
            delta = sp0 - w0
            pltpu.sync_copy(sp_hbm.at[pl.ds(w0, 24)], sp_v)
            idx_a = jnp.where(iota16 < delta, _P + iota16, iota16 - delta)
            rel_b = iota16 + 8 - delta
            idx_b = jnp.where(rel_b < _P, rel_b, iota16 + 8)
            cpa = pltpu.async_copy(
                sp_v.at[pl.ds(0, 16)], out_b.at[idx_a], psem)
            cpb = pltpu.async_copy(
                sp_v.at[pl.ds(8, 16)], out_b.at[idx_b], psem)
            cpa.wait()
            cpb.wait()

        gsem = (gsem0, gsem1)
        osem = (osem0, osem1)
        row0 = _P + kk * _TOK_W

        def start_gather(j, p):
            return pltpu.async_copy(
                w_hbm.at[idx_v.at[pl.ds(j * _CHUNK, _CHUNK)]],
                rows_v.at[p], gsem[p])

        gd = [None, None]
        wd = [None, None]
        gd[0] = start_gather(0, 0)
        for j in range(_NCHUNK):
            p = j % 2
            q = 1 - p
            gd[p].wait()
            if j >= 1:
                for d in wd[q]:
                    d.wait()
            if j + 1 < _NCHUNK:
                gd[q] = start_gather(j + 1, q)
            wd[p] = [
                pltpu.async_copy(
                    rows_v.at[p, pl.ds(16 * s, 16)],
                    out_b.at[iota16 + (row0 + j * _CHUNK + 16 * s)],
                    osem[p])
                for s in range(2)
            ]
        for d in wd[_NCHUNK % 2 ^ 1]:
            d.wait()

    return k(tokens_flat, soft_prompts, W)


def kernel(tokens, soft_prompts, W):
    tokens_flat = tokens.reshape(-1).astype(jnp.int32)
    sp_flat = soft_prompts.reshape(_B * _P, _D)
    return _embed_concat(tokens_flat, sp_flat, W)


# speedup vs baseline: 7.4057x; 2.0307x over previous
import functools

import jax
import jax.numpy as jnp
from jax import lax
from jax.experimental import pallas as pl
from jax.experimental.pallas import tpu as pltpu
from jax.experimental.pallas import tpu_sc as plsc

_D = 1024
_B = 4
_S = 2048
_P = 20
_R = _S + _P
_NDT = _D // 128

_NC = 2
_NS = 16
_NW = _NC * _NS
_SEQ_W = _S // _NW
_CSEQ = 4
_NCHUNK = _SEQ_W // _CSEQ
_CPIECE = _CSEQ * _B * _NDT
_SPROWS = _B * _P * _NDT
_OUTROWS = _R * _NDT * _B
_WROWS = (100000 // 8) * 8 * _NDT


def _embed_concat(tokens_flat, sp_pieces, w_pieces):
    mesh = plsc.VectorSubcoreMesh(core_axis_name="c", subcore_axis_name="s")

    @functools.partial(
        pl.kernel,
        mesh=mesh,
        out_type=jax.ShapeDtypeStruct((_OUTROWS, 128), jnp.float32),
        scratch_types=[
            pltpu.VMEM((_B * _SEQ_W,), jnp.int32),
            pltpu.VMEM((2, _CPIECE, 128), jnp.float32),
            pltpu.VMEM((_SPROWS // 8, 128), jnp.float32),
            pltpu.SemaphoreType.DMA,
            pltpu.SemaphoreType.DMA,
            pltpu.SemaphoreType.DMA,
            pltpu.SemaphoreType.DMA,
            pltpu.SemaphoreType.DMA,
        ],
        compiler_params=pltpu.CompilerParams(needs_layout_passes=False),
    )
    def k(tok_hbm, sp_hbm, w_hbm, out_hbm, idx_v, rows_v, sp_v,
          gsem0, gsem1, osem0, osem1, psem):
        wid = lax.axis_index("s") * _NC + lax.axis_index("c")
        seq0 = pl.multiple_of(wid * _SEQ_W, _SEQ_W)
        iota16 = lax.iota(jnp.int32, 16)

        for bb in range(_B):
            pltpu.sync_copy(
                tok_hbm.at[pl.ds(pl.multiple_of(bb * _S + seq0, _SEQ_W),
                                 _SEQ_W)],
                idx_v.at[pl.ds(bb * _SEQ_W, _SEQ_W)])

        @pl.when(wid < 8)
        def _():
            off = pl.multiple_of(wid * (_SPROWS // 8), 8)
            pltpu.sync_copy(sp_hbm.at[pl.ds(off, _SPROWS // 8)], sp_v)
            pltpu.async_copy(
                sp_v, out_hbm.at[pl.ds(off, _SPROWS // 8)], psem).wait()

        gsem = (gsem0, gsem1)
        osem = (osem0, osem1)

        def issue_gathers(c, p):
            for v in range(_CPIECE // 16):
                pp = 16 * v + iota16
                sl = lax.shift_right_logical(pp, 5)
                dt = lax.bitwise_and(lax.shift_right_logical(pp, 2), 7)
                bb = lax.bitwise_and(pp, 3)
                t = plsc.load_gather(
                    idx_v, [bb * _SEQ_W + c * _CSEQ + sl])
                gidx = (lax.shift_right_logical(t, 3) * (8 * _NDT)
                        + dt * 8 + lax.bitwise_and(t, 7))
                pltpu.async_copy(
                    w_hbm.at[gidx], rows_v.at[p, pl.ds(16 * v, 16)],
                    gsem[p])

        def drain_gathers(p):
            pltpu.make_async_copy(
                w_hbm.at[pl.ds(0, _CPIECE)], rows_v.at[p], gsem[p]).wait()

        def issue_write(c, p):
            row = pl.multiple_of(
                (_P + seq0 + c * _CSEQ) * (_NDT * _B), _CPIECE)
            pltpu.async_copy(
                rows_v.at[p], out_hbm.at[pl.ds(row, _CPIECE)], osem[p])

        def drain_write(p):
            pltpu.make_async_copy(
                rows_v.at[p], out_hbm.at[pl.ds(0, _CPIECE)], osem[p]).wait()

        issue_gathers(0, 0)

        def body(i2, carry):
            for h in range(2):
                c = 2 * i2 + h
                p = h
                q = 1 - p
                drain_gathers(p)
                issue_write(c, p)

                @pl.when(c >= 1)
                def _():
                    drain_write(q)

                @pl.when(c < _NCHUNK - 1)
                def _():
                    issue_gathers(c + 1, q)
            return carry

        lax.fori_loop(0, _NCHUNK // 2, body, 0)
        drain_write(1)

    return k(tokens_flat, sp_pieces, w_pieces)


def kernel(tokens, soft_prompts, W):
    tokens_flat = tokens.reshape(-1).astype(jnp.int32)
    sp_pieces = (soft_prompts.reshape(_B, _P, _NDT, 128)
                 .transpose(1, 2, 0, 3).reshape(_SPROWS, 128))
    w_pieces = (W.reshape(_WROWS // 64, 8, _NDT, 128)
                .transpose(0, 2, 1, 3).reshape(_WROWS, 128))
    out = _embed_concat(tokens_flat, sp_pieces, w_pieces)
    return (out.reshape(_R, _NDT, _B, 128)
            .transpose(2, 0, 1, 3).reshape(_B, _R, _D))


# speedup vs baseline: 8.1238x; 1.0970x over previous
import functools

import jax
import jax.numpy as jnp
from jax import lax
from jax.experimental import pallas as pl
from jax.experimental.pallas import tpu as pltpu
from jax.experimental.pallas import tpu_sc as plsc

_D = 1024
_B = 4
_S = 2048
_P = 20
_R = _S + _P
_NDT = _D // 128

_NC = 2
_NS = 16
_NW = _NC * _NS
_SEQ_W = _S // _NW
_CSEQ = 2
_NCHUNK = _SEQ_W // _CSEQ
_NBUF = 4
_CPIECE = _CSEQ * _B * _NDT
_SPROWS = _B * _P * _NDT
_OUTROWS = _R * _NDT * _B
_WROWS = (100000 // 8) * 8 * _NDT


def _embed_concat(tokens_flat, sp_pieces, w_pieces):
    mesh = plsc.VectorSubcoreMesh(core_axis_name="c", subcore_axis_name="s")

    @functools.partial(
        pl.kernel,
        mesh=mesh,
        out_type=jax.ShapeDtypeStruct((_OUTROWS, 128), jnp.float32),
        scratch_types=[
            pltpu.VMEM((_B * _SEQ_W,), jnp.int32),
            pltpu.VMEM((_NBUF, _CPIECE, 128), jnp.float32),
            pltpu.VMEM((_SPROWS // 8, 128), jnp.float32),
            pltpu.SemaphoreType.DMA,
            pltpu.SemaphoreType.DMA,
            pltpu.SemaphoreType.DMA,
            pltpu.SemaphoreType.DMA,
            pltpu.SemaphoreType.DMA,
            pltpu.SemaphoreType.DMA,
            pltpu.SemaphoreType.DMA,
            pltpu.SemaphoreType.DMA,
            pltpu.SemaphoreType.DMA,
        ],
        compiler_params=pltpu.CompilerParams(needs_layout_passes=False),
    )
    def k(tok_hbm, sp_hbm, w_hbm, out_hbm, idx_v, rows_v, sp_v,
          gsem0, gsem1, gsem2, gsem3, osem0, osem1, osem2, osem3, psem):
        wid = lax.axis_index("s") * _NC + lax.axis_index("c")
        seq0 = pl.multiple_of(wid * _SEQ_W, _SEQ_W)
        iota16 = lax.iota(jnp.int32, 16)

        for bb in range(_B):
            pltpu.sync_copy(
                tok_hbm.at[pl.ds(pl.multiple_of(bb * _S + seq0, _SEQ_W),
                                 _SEQ_W)],
                idx_v.at[pl.ds(bb * _SEQ_W, _SEQ_W)])

        @pl.when(wid < 8)
        def _():
            off = pl.multiple_of(wid * (_SPROWS // 8), 8)
            pltpu.sync_copy(sp_hbm.at[pl.ds(off, _SPROWS // 8)], sp_v)
            pltpu.async_copy(
                sp_v, out_hbm.at[pl.ds(off, _SPROWS // 8)], psem).wait()

        gsem = (gsem0, gsem1, gsem2, gsem3)
        osem = (osem0, osem1, osem2, osem3)

        def issue_gathers(c, p):
            for v in range(_CPIECE // 16):
                pp = 16 * v + iota16
                sl = lax.shift_right_logical(pp, 5)
                dt = lax.bitwise_and(lax.shift_right_logical(pp, 2), 7)
                bb = lax.bitwise_and(pp, 3)
                t = plsc.load_gather(
                    idx_v, [bb * _SEQ_W + c * _CSEQ + sl])
                gidx = (lax.shift_right_logical(t, 3) * (8 * _NDT)
                        + dt * 8 + lax.bitwise_and(t, 7))
                pltpu.async_copy(
                    w_hbm.at[gidx], rows_v.at[p, pl.ds(16 * v, 16)],
                    gsem[p])

        def drain_gathers(p):
            pltpu.make_async_copy(
                w_hbm.at[pl.ds(0, _CPIECE)], rows_v.at[p], gsem[p]).wait()

        def issue_write(c, p):
            row = pl.multiple_of(
                (_P + seq0 + c * _CSEQ) * (_NDT * _B), _CPIECE)
            pltpu.async_copy(
                rows_v.at[p], out_hbm.at[pl.ds(row, _CPIECE)], osem[p])

        def drain_write(p):
            pltpu.make_async_copy(
                rows_v.at[p], out_hbm.at[pl.ds(0, _CPIECE)], osem[p]).wait()

        issue_gathers(0, 0)
        issue_gathers(1, 1)

        def body(i2, carry):
            for h in range(_NBUF):
                c = _NBUF * i2 + h
                p = h
                q = (h + 2) % _NBUF
                drain_gathers(p)
                issue_write(c, p)

                @pl.when(c >= 2)
                def _():
                    drain_write(q)

                @pl.when(c < _NCHUNK - 2)
                def _():
                    issue_gathers(c + 2, q)
            return carry

        lax.fori_loop(0, _NCHUNK // _NBUF, body, 0)
        drain_write((_NCHUNK - 2) % _NBUF)
        drain_write((_NCHUNK - 1) % _NBUF)

    return k(tokens_flat, sp_pieces, w_pieces)


def kernel(tokens, soft_prompts, W):
    tokens_flat = tokens.reshape(-1).astype(jnp.int32)
    sp_pieces = (soft_prompts.reshape(_B, _P, _NDT, 128)
                 .transpose(1, 2, 0, 3).reshape(_SPROWS, 128))
    w_pieces = (W.reshape(_WROWS // 64, 8, _NDT, 128)
                .transpose(0, 2, 1, 3).reshape(_WROWS, 128))
    out = _embed_concat(tokens_flat, sp_pieces, w_pieces)
    return (out.reshape(_R, _NDT, _B, 128)
            .transpose(2, 0, 1, 3).reshape(_B, _R, _D))


# speedup vs baseline: 8.6588x; 1.0659x over previous
import functools

import jax
import jax.numpy as jnp
from jax import lax
from jax.experimental import pallas as pl
from jax.experimental.pallas import tpu as pltpu
from jax.experimental.pallas import tpu_sc as plsc

_D = 1024
_B = 4
_S = 2048
_P = 20
_R = _S + _P
_NDT = _D // 128

_NC = 2
_NS = 16
_NW = _NC * _NS
_SEQ_W = _S // _NW
_CSEQ = 4
_NCHUNK = _SEQ_W // _CSEQ
_NBUF = 4
_CPIECE = _CSEQ * _B * _NDT
_SPROWS = _B * _P * _NDT
_OUTROWS = _R * _NDT * _B
_WROWS = (100000 // 8) * 8 * _NDT


def _embed_concat(tokens_flat, sp_pieces, w_pieces):
    mesh = plsc.VectorSubcoreMesh(core_axis_name="c", subcore_axis_name="s")

    @functools.partial(
        pl.kernel,
        mesh=mesh,
        out_type=jax.ShapeDtypeStruct((_OUTROWS, 128), jnp.float32),
        scratch_types=[
            pltpu.VMEM((_B * _SEQ_W,), jnp.int32),
            pltpu.VMEM((_NBUF, _CPIECE, 128), jnp.float32),
            pltpu.VMEM((_SPROWS // 8, 128), jnp.float32),
            pltpu.SemaphoreType.DMA,
            pltpu.SemaphoreType.DMA,
            pltpu.SemaphoreType.DMA,
            pltpu.SemaphoreType.DMA,
            pltpu.SemaphoreType.DMA,
            pltpu.SemaphoreType.DMA,
            pltpu.SemaphoreType.DMA,
            pltpu.SemaphoreType.DMA,
            pltpu.SemaphoreType.DMA,
        ],
        compiler_params=pltpu.CompilerParams(needs_layout_passes=False),
    )
    def k(tok_hbm, sp_hbm, w_hbm, out_hbm, idx_v, rows_v, sp_v,
          gsem0, gsem1, gsem2, gsem3, osem0, osem1, osem2, osem3, psem):
        wid = lax.axis_index("s") * _NC + lax.axis_index("c")
        seq0 = pl.multiple_of(wid * _SEQ_W, _SEQ_W)
        iota16 = lax.iota(jnp.int32, 16)

        for bb in range(_B):
            pltpu.sync_copy(
                tok_hbm.at[pl.ds(pl.multiple_of(bb * _S + seq0, _SEQ_W),
                                 _SEQ_W)],
                idx_v.at[pl.ds(bb * _SEQ_W, _SEQ_W)])

        @pl.when(wid < 8)
        def _():
            off = pl.multiple_of(wid * (_SPROWS // 8), 8)
            pltpu.sync_copy(sp_hbm.at[pl.ds(off, _SPROWS // 8)], sp_v)
            pltpu.async_copy(
                sp_v, out_hbm.at[pl.ds(off, _SPROWS // 8)], psem).wait()

        gsem = (gsem0, gsem1, gsem2, gsem3)
        osem = (osem0, osem1, osem2, osem3)

        def issue_gathers(c, p):
            for v in range(_CPIECE // 16):
                pp = 16 * v + iota16
                sl = lax.shift_right_logical(pp, 5)
                dt = lax.bitwise_and(lax.shift_right_logical(pp, 2), 7)
                bb = lax.bitwise_and(pp, 3)
                t = plsc.load_gather(
                    idx_v, [bb * _SEQ_W + c * _CSEQ + sl])
                gidx = (lax.shift_right_logical(t, 3) * (8 * _NDT)
                        + dt * 8 + lax.bitwise_and(t, 7))
                pltpu.async_copy(
                    w_hbm.at[gidx], rows_v.at[p, pl.ds(16 * v, 16)],
                    gsem[p])

        def drain_gathers(p):
            pltpu.make_async_copy(
                w_hbm.at[pl.ds(0, _CPIECE)], rows_v.at[p], gsem[p]).wait()

        def issue_write(c, p):
            row = pl.multiple_of(
                (_P + seq0 + c * _CSEQ) * (_NDT * _B), _CPIECE)
            pltpu.async_copy(
                rows_v.at[p], out_hbm.at[pl.ds(row, _CPIECE)], osem[p])

        def drain_write(p):
            pltpu.make_async_copy(
                rows_v.at[p], out_hbm.at[pl.ds(0, _CPIECE)], osem[p]).wait()

        def body(i2, carry):
            for h in range(_NBUF):
                g = _NBUF * i2 + h
                q = (h + 2) % _NBUF

                @pl.when(g >= _NBUF)
                def _():
                    drain_write(h)

                @pl.when(g < _NCHUNK)
                def _():
                    issue_gathers(g, h)

                @pl.when(jnp.logical_and(g >= 2, g < _NCHUNK + 2))
                def _():
                    drain_gathers(q)
                    issue_write(g - 2, q)
            return carry

        lax.fori_loop(0, (_NCHUNK + _NBUF) // _NBUF, body, 0)

    return k(tokens_flat, sp_pieces, w_pieces)


def kernel(tokens, soft_prompts, W):
    tokens_flat = tokens.reshape(-1).astype(jnp.int32)
    sp_pieces = (soft_prompts.reshape(_B, _P, _NDT, 128)
                 .transpose(1, 2, 0, 3).reshape(_SPROWS, 128))
    w_pieces = (W.reshape(_WROWS // 64, 8, _NDT, 128)
                .transpose(0, 2, 1, 3).reshape(_WROWS, 128))
    out = _embed_concat(tokens_flat, sp_pieces, w_pieces)
    return (out.reshape(_R, _NDT, _B, 128)
            .transpose(2, 0, 1, 3).reshape(_B, _R, _D))
